# Initial kernel scaffold; baseline (speedup 1.0000x reference)
import jax
import jax.numpy as jnp
from jax.experimental import pallas as pl


def kernel(x, edge_index, edge_weight, h, params):
    raise NotImplementedError("write your pallas kernel here")


# speedup vs baseline: 8.9309x; 8.9309x over previous
import functools

import jax
import jax.numpy as jnp
from jax import lax
from jax.experimental import pallas as pl
from jax.experimental.pallas import tpu as pltpu
from jax.experimental.pallas import tpu_sc as plsc

N = 10000
E = 320000
D = 128
NC = 2
NS = 16
CH = 80
SCH = 25

NSUP = 10
NSUP2 = 5
NPAD = 10240
RPT = NPAD // NS


def _rsqrt_sc(d):
    i = lax.bitcast_convert_type(d, jnp.int32)
    y = lax.bitcast_convert_type(jnp.int32(0x5F3759DF) - (i >> 1), jnp.float32)
    for _ in range(3):
        t = (d * y) * y
        y = y * (1.5 - 0.5 * t)
    return jnp.where(d > 0.0, y, 0.0)


def _scaled_scatter(table, rsup, csup, lwsup, rows_v, acc, sem, cc):
    pltpu.async_copy(table.at[rsup.at[cc]], rows_v, sem).wait()

    def scale_body(e, _):
        lwb = plsc.load_gather(
            lwsup, [jnp.broadcast_to(cc, (16,)), jnp.broadcast_to(e, (16,))])
        for j in range(8):
            sl = pl.ds(j * 16, 16)
            rows_v[e, sl] = rows_v[e, sl] * lwb
        return 0

    lax.fori_loop(0, CH, scale_body, 0)
    pltpu.sync_copy(rows_v, acc.at[csup.at[cc]], add=True)


def _conv_xh_body(x_hbm, h_hbm, r4, c4, e4, zf, z1, sx_o, sh_o, lw_o,
                  rsup, csup, esup, lwsup, dis_v, dtmp, rows_v, acc, deg_sp,
                  sem):
    c = lax.axis_index("c")
    s = lax.axis_index("s")

    pltpu.sync_copy(z1.at[pl.ds(s * RPT, RPT)], deg_sp.at[pl.ds(s * RPT, RPT)])
    plsc.subcore_barrier()

    def deg_sup(sup, _):
        pltpu.sync_copy(r4.at[s, sup], rsup)
        pltpu.sync_copy(c4.at[s, sup], csup)
        pltpu.sync_copy(e4.at[s, sup], esup)

        def deg_ch(cc, _):
            for g in range(5):
                sl = pl.ds(g * 16, 16)
                esup[cc, sl] = jnp.where(
                    rsup[cc, sl] == csup[cc, sl], 0.0, esup[cc, sl])
            pltpu.sync_copy(esup.at[cc], deg_sp.at[rsup.at[cc]], add=True)
            return 0

        lax.fori_loop(0, SCH, deg_ch, 0)
        return 0

    lax.fori_loop(0, NSUP, deg_sup, 0)
    plsc.subcore_barrier()

    def dis_outer(k, _):
        pltpu.sync_copy(deg_sp.at[pl.ds(k * RPT, RPT)], dtmp)

        def dis_inner(v, _):
            d = dtmp[pl.ds(v * 16, 16)]
            dis_v[pl.ds(k * RPT + v * 16, 16)] = _rsqrt_sc(d)
            return 0

        lax.fori_loop(0, RPT // 16, dis_inner, 0)
        return 0

    lax.fori_loop(0, NS, dis_outer, 0)

    pltpu.sync_copy(zf.at[pl.ds(s * RPT, RPT)], acc.at[pl.ds(s * RPT, RPT)])
    plsc.subcore_barrier()

    def make_pass(table):
        def sup_body(sup, _):
            pltpu.sync_copy(r4.at[s, sup], rsup)
            pltpu.sync_copy(c4.at[s, sup], csup)
            pltpu.sync_copy(e4.at[s, sup], esup)

            def ch_body(cc, _):
                for g in range(5):
                    sl = pl.ds(g * 16, 16)
                    r = rsup[cc, sl]
                    co = csup[cc, sl]
                    mew = jnp.where(r == co, 0.0, esup[cc, sl])
                    dr = plsc.load_gather(dis_v, [r])
                    dc = plsc.load_gather(dis_v, [co])
                    lwsup[cc, sl] = -(dr * mew * dc)
                _scaled_scatter(table, rsup, csup, lwsup, rows_v, acc, sem, cc)
                return 0

            lax.fori_loop(0, SCH, ch_body, 0)

            @pl.when(c == 0)
            def _():
                pltpu.sync_copy(lwsup, lw_o.at[s, sup])

            return 0

        lax.fori_loop(0, NSUP, sup_body, 0)

    @pl.when(c == 0)
    def _():
        make_pass(x_hbm)

    @pl.when(c == 1)
    def _():
        make_pass(h_hbm)

    plsc.subcore_barrier()
    sl_out = pl.ds(s * RPT, RPT)

    @pl.when(c == 0)
    def _():
        pltpu.sync_copy(acc.at[sl_out], sx_o.at[sl_out])

    @pl.when(c == 1)
    def _():
        pltpu.sync_copy(acc.at[sl_out], sh_o.at[sl_out])


def _conv_hr_body(hr_hbm, r4, c4, lw4, zf, shr_o,
                  rsup, csup, lwsup, rows_v, acc, sem):
    c = lax.axis_index("c")
    s = lax.axis_index("s")
    w = c * NS + s

    pltpu.sync_copy(zf.at[pl.ds(s * RPT, RPT)], acc.at[pl.ds(s * RPT, RPT)])
    plsc.subcore_barrier()

    def sup_body(sup, _):
        pltpu.sync_copy(r4.at[w, sup], rsup)
        pltpu.sync_copy(c4.at[w, sup], csup)
        pltpu.sync_copy(lw4.at[w, sup], lwsup)

        def ch_body(cc, _):
            _scaled_scatter(hr_hbm, rsup, csup, lwsup, rows_v, acc, sem, cc)
            return 0

        lax.fori_loop(0, SCH, ch_body, 0)
        return 0

    lax.fori_loop(0, NSUP2, sup_body, 0)
    plsc.subcore_barrier()
    pltpu.sync_copy(acc.at[pl.ds(s * RPT, RPT)],
                    shr_o.at[pl.ds(c * NPAD + s * RPT, RPT)])


_conv_xh = functools.partial(
    pl.kernel,
    out_type=(
        jax.ShapeDtypeStruct((NPAD, D), jnp.float32),
        jax.ShapeDtypeStruct((NPAD, D), jnp.float32),
        jax.ShapeDtypeStruct((NS, NSUP, SCH, CH), jnp.float32),
    ),
    mesh=plsc.VectorSubcoreMesh(core_axis_name="c", subcore_axis_name="s"),
    compiler_params=pltpu.CompilerParams(needs_layout_passes=False),
    scratch_types=[
        pltpu.VMEM((SCH, CH), jnp.int32),
        pltpu.VMEM((SCH, CH), jnp.int32),
        pltpu.VMEM((SCH, CH), jnp.float32),
        pltpu.VMEM((SCH, CH), jnp.float32),
        pltpu.VMEM((NPAD,), jnp.float32),
        pltpu.VMEM((RPT,), jnp.float32),
        pltpu.VMEM((CH, D), jnp.float32),
        pltpu.VMEM_SHARED((NPAD, D), jnp.float32),
        pltpu.VMEM_SHARED((NPAD,), jnp.float32),
        pltpu.SemaphoreType.DMA,
    ],
)(_conv_xh_body)


_conv_hr = functools.partial(
    pl.kernel,
    out_type=jax.ShapeDtypeStruct((NC * NPAD, D), jnp.float32),
    mesh=plsc.VectorSubcoreMesh(core_axis_name="c", subcore_axis_name="s"),
    compiler_params=pltpu.CompilerParams(needs_layout_passes=False),
    scratch_types=[
        pltpu.VMEM((SCH, CH), jnp.int32),
        pltpu.VMEM((SCH, CH), jnp.int32),
        pltpu.VMEM((SCH, CH), jnp.float32),
        pltpu.VMEM((CH, D), jnp.float32),
        pltpu.VMEM_SHARED((NPAD, D), jnp.float32),
        pltpu.SemaphoreType.DMA,
    ],
)(_conv_hr_body)


def _gates_body(x_ref, h_ref, sx_ref, sh_ref, wx_ref, w1x_ref, wh_ref,
                w1h_ref, b_ref, z_ref, hr_ref, ph_ref):
    x = x_ref[...]
    h = h_ref[...]
    f32 = jnp.float32
    a = (jnp.dot(x, wx_ref[...], preferred_element_type=f32)
         + jnp.dot(sx_ref[...], w1x_ref[...], preferred_element_type=f32)
         + b_ref[...])
    bm = (jnp.dot(h, wh_ref[...], preferred_element_type=f32)
          + jnp.dot(sh_ref[...], w1h_ref[...], preferred_element_type=f32))
    z = jax.nn.sigmoid(a[:, :D] + bm[:, :D])
    r = jax.nn.sigmoid(a[:, D:2 * D] + bm[:, D:2 * D])
    z_ref[...] = z
    hr_ref[...] = h * r
    ph_ref[...] = a[:, 2 * D:]


def _out_body(ph_ref, hr_ref, s0_ref, s1_ref, z_ref, h_ref, whh_ref,
              w1hh_ref, wlin_ref, bhh_ref, blin_ref, out_ref, h0_ref):
    f32 = jnp.float32
    shr = s0_ref[...] + s1_ref[...]
    ht = jnp.tanh(ph_ref[...]
                  + jnp.dot(hr_ref[...], whh_ref[...], preferred_element_type=f32)
                  + jnp.dot(shr, w1hh_ref[...], preferred_element_type=f32)
                  + bhh_ref[...])
    z = z_ref[...]
    h0 = z * h_ref[...] + (1.0 - z) * ht
    h0_ref[...] = h0
    out_ref[...] = (jnp.dot(jnp.maximum(h0, 0.0), wlin_ref[...],
                            preferred_element_type=f32) + blin_ref[...])


_RB = 1000


def _row_spec(ncols):
    return pl.BlockSpec((_RB, ncols), lambda i: (i, 0))


def _full_spec(shape):
    nd = len(shape)
    return pl.BlockSpec(shape, lambda i: (0,) * nd)


def kernel(x, edge_index, edge_weight, h, params):
    p = params
    row = edge_index[0]
    col = edge_index[1]
    r4 = row.reshape(NS, NSUP, SCH, CH)
    c4 = col.reshape(NS, NSUP, SCH, CH)
    e4 = edge_weight.reshape(NS, NSUP, SCH, CH).astype(jnp.float32)
    zf = jnp.zeros((NPAD, D), jnp.float32)
    z1 = jnp.zeros((NPAD,), jnp.float32)

    sx, sh, lw = _conv_xh(x, h, r4, c4, e4, zf, z1)

    wx = jnp.concatenate([p["W0_xz"], p["W0_xr"], p["W0_xh"]], axis=1)
    w1x = jnp.concatenate([p["W1_xz"], p["W1_xr"], p["W1_xh"]], axis=1)
    wh = jnp.concatenate([p["W0_hz"], p["W0_hr"]], axis=1)
    w1h = jnp.concatenate([p["W1_hz"], p["W1_hr"]], axis=1)
    b384 = jnp.concatenate(
        [p["b_xz"] + p["b_hz"], p["b_xr"] + p["b_hr"], p["b_xh"]]).reshape(1, 3 * D)

    z, hr, ph = pl.pallas_call(
        _gates_body,
        grid=(N // _RB,),
        in_specs=[
            _row_spec(D), _row_spec(D), _row_spec(D), _row_spec(D),
            _full_spec((D, 3 * D)), _full_spec((D, 3 * D)),
            _full_spec((D, 2 * D)), _full_spec((D, 2 * D)),
            _full_spec((1, 3 * D)),
        ],
        out_specs=[_row_spec(D), _row_spec(D), _row_spec(D)],
        out_shape=[
            jax.ShapeDtypeStruct((N, D), jnp.float32),
            jax.ShapeDtypeStruct((N, D), jnp.float32),
            jax.ShapeDtypeStruct((N, D), jnp.float32),
        ],
    )(x, h, sx[:N], sh[:N], wx, w1x, wh, w1h, b384)

    shrp = _conv_hr(hr, row.reshape(NC * NS, NSUP2, SCH, CH),
                    col.reshape(NC * NS, NSUP2, SCH, CH),
                    lw.reshape(NC * NS, NSUP2, SCH, CH), zf)

    bhh = p["b_hh"].reshape(1, D)
    blin = p["b_lin"].reshape(1, D)
    out, h0 = pl.pallas_call(
        _out_body,
        grid=(N // _RB,),
        in_specs=[
            _row_spec(D), _row_spec(D), _row_spec(D), _row_spec(D),
            _row_spec(D), _row_spec(D),
            _full_spec((D, D)), _full_spec((D, D)), _full_spec((D, D)),
            _full_spec((1, D)), _full_spec((1, D)),
        ],
        out_specs=[_row_spec(D), _row_spec(D)],
        out_shape=[
            jax.ShapeDtypeStruct((N, D), jnp.float32),
            jax.ShapeDtypeStruct((N, D), jnp.float32),
        ],
    )(ph, hr, shrp[:N], shrp[NPAD:NPAD + N], z, h, p["W0_hh"], p["W1_hh"],
      p["W_lin"], bhh, blin)
    return (out, h0)


# speedup vs baseline: 14.0379x; 1.5718x over previous
import functools

import jax
import jax.numpy as jnp
from jax import lax
from jax.experimental import pallas as pl
from jax.experimental.pallas import tpu as pltpu
from jax.experimental.pallas import tpu_sc as plsc

N = 10000
E = 320000
D = 128
NC = 2
NS = 16
CH = 80
SCH = 25

NSUP = 10
NSUP2 = 5
NPAD = 10240
RPT = NPAD // NS


def _rsqrt_sc(d):
    i = lax.bitcast_convert_type(d, jnp.int32)
    y = lax.bitcast_convert_type(jnp.int32(0x5F3759DF) - (i >> 1), jnp.float32)
    for _ in range(3):
        t = (d * y) * y
        y = y * (1.5 - 0.5 * t)
    return jnp.where(d > 0.0, y, 0.0)


def _scale_rows(rows2, lwsup, par, cc):
    def scale_body(e, _):
        lwb = plsc.load_gather(
            lwsup, [jnp.broadcast_to(cc, (16,)), jnp.broadcast_to(e, (16,))])
        for j in range(8):
            sl = pl.ds(j * 16, 16)
            rows2[par, e, sl] = rows2[par, e, sl] * lwb
        return 0

    lax.fori_loop(0, CH, scale_body, 0)


def _spmm_super(table, rsup, csup, lwsup, rows2, acc, sem, compute_lw):
    pltpu.async_copy(table.at[rsup.at[0]], rows2.at[0], sem)

    def ch_body(cc, _):
        par = lax.rem(cc, 2)
        pltpu.make_async_copy(table.at[rsup.at[cc]], rows2.at[par], sem).wait()

        @pl.when(cc < SCH - 1)
        def _():
            pltpu.async_copy(table.at[rsup.at[cc + 1]],
                             rows2.at[lax.rem(cc + 1, 2)], sem)

        compute_lw(cc)
        _scale_rows(rows2, lwsup, par, cc)
        pltpu.sync_copy(rows2.at[par], acc.at[csup.at[cc]], add=True)
        return 0

    lax.fori_loop(0, SCH, ch_body, 0)


def _conv_xh_body(x_hbm, h_hbm, r4, c4, e4, zf, z1, sx_o, sh_o, lw_o,
                  rsup, csup, esup, lwsup, dis_v, dtmp, rows2, acc, deg_sp,
                  sem, dsem):
    c = lax.axis_index("c")
    s = lax.axis_index("s")

    pltpu.sync_copy(z1.at[pl.ds(s * RPT, RPT)], deg_sp.at[pl.ds(s * RPT, RPT)])
    plsc.subcore_barrier()

    def deg_sup(sup, _):
        pltpu.sync_copy(r4.at[s, sup], rsup)
        pltpu.sync_copy(c4.at[s, sup], csup)
        pltpu.sync_copy(e4.at[s, sup], esup)

        def deg_ch(cc, _):
            for g in range(5):
                sl = pl.ds(g * 16, 16)
                esup[cc, sl] = jnp.where(
                    rsup[cc, sl] == csup[cc, sl], 0.0, esup[cc, sl])
            pltpu.async_copy(esup.at[cc], deg_sp.at[rsup.at[cc]], dsem,
                             add=True)
            return 0

        lax.fori_loop(0, SCH, deg_ch, 0)

        def deg_drain(cc, _):
            pltpu.make_async_copy(esup.at[cc], deg_sp.at[rsup.at[cc]],
                                  dsem).wait()
            return 0

        lax.fori_loop(0, SCH, deg_drain, 0)
        return 0

    lax.fori_loop(0, NSUP, deg_sup, 0)
    plsc.subcore_barrier()

    def dis_outer(k, _):
        pltpu.sync_copy(deg_sp.at[pl.ds(k * RPT, RPT)], dtmp)

        def dis_inner(v, _):
            d = dtmp[pl.ds(v * 16, 16)]
            dis_v[pl.ds(k * RPT + v * 16, 16)] = _rsqrt_sc(d)
            return 0

        lax.fori_loop(0, RPT // 16, dis_inner, 0)
        return 0

    lax.fori_loop(0, NS, dis_outer, 0)

    pltpu.sync_copy(zf.at[pl.ds(s * RPT, RPT)], acc.at[pl.ds(s * RPT, RPT)])
    plsc.subcore_barrier()

    def make_pass(table):
        def sup_body(sup, _):
            pltpu.sync_copy(r4.at[s, sup], rsup)
            pltpu.sync_copy(c4.at[s, sup], csup)
            pltpu.sync_copy(e4.at[s, sup], esup)

            def compute_lw(cc):
                for g in range(5):
                    sl = pl.ds(g * 16, 16)
                    r = rsup[cc, sl]
                    co = csup[cc, sl]
                    mew = jnp.where(r == co, 0.0, esup[cc, sl])
                    dr = plsc.load_gather(dis_v, [r])
                    dc = plsc.load_gather(dis_v, [co])
                    lwsup[cc, sl] = -(dr * mew * dc)

            _spmm_super(table, rsup, csup, lwsup, rows2, acc, sem, compute_lw)

            @pl.when(c == 0)
            def _():
                pltpu.sync_copy(lwsup, lw_o.at[s, sup])

            return 0

        lax.fori_loop(0, NSUP, sup_body, 0)

    @pl.when(c == 0)
    def _():
        make_pass(x_hbm)

    @pl.when(c == 1)
    def _():
        make_pass(h_hbm)

    plsc.subcore_barrier()
    sl_out = pl.ds(s * RPT, RPT)

    @pl.when(c == 0)
    def _():
        pltpu.sync_copy(acc.at[sl_out], sx_o.at[sl_out])

    @pl.when(c == 1)
    def _():
        pltpu.sync_copy(acc.at[sl_out], sh_o.at[sl_out])


def _conv_hr_body(hr_hbm, r4, c4, lw4, zf, shr_o,
                  rsup, csup, lwsup, rows2, acc, sem):
    c = lax.axis_index("c")
    s = lax.axis_index("s")
    w = c * NS + s

    pltpu.sync_copy(zf.at[pl.ds(s * RPT, RPT)], acc.at[pl.ds(s * RPT, RPT)])
    plsc.subcore_barrier()

    def sup_body(sup, _):
        pltpu.sync_copy(r4.at[w, sup], rsup)
        pltpu.sync_copy(c4.at[w, sup], csup)
        pltpu.sync_copy(lw4.at[w, sup], lwsup)
        _spmm_super(hr_hbm, rsup, csup, lwsup, rows2, acc, sem,
                    lambda cc: None)
        return 0

    lax.fori_loop(0, NSUP2, sup_body, 0)
    plsc.subcore_barrier()
    pltpu.sync_copy(acc.at[pl.ds(s * RPT, RPT)],
                    shr_o.at[pl.ds(c * NPAD + s * RPT, RPT)])


_conv_xh = functools.partial(
    pl.kernel,
    out_type=(
        jax.ShapeDtypeStruct((NPAD, D), jnp.float32),
        jax.ShapeDtypeStruct((NPAD, D), jnp.float32),
        jax.ShapeDtypeStruct((NS, NSUP, SCH, CH), jnp.float32),
    ),
    mesh=plsc.VectorSubcoreMesh(core_axis_name="c", subcore_axis_name="s"),
    compiler_params=pltpu.CompilerParams(needs_layout_passes=False),
    scratch_types=[
        pltpu.VMEM((SCH, CH), jnp.int32),
        pltpu.VMEM((SCH, CH), jnp.int32),
        pltpu.VMEM((SCH, CH), jnp.float32),
        pltpu.VMEM((SCH, CH), jnp.float32),
        pltpu.VMEM((NPAD,), jnp.float32),
        pltpu.VMEM((RPT,), jnp.float32),
        pltpu.VMEM((2, CH, D), jnp.float32),
        pltpu.VMEM_SHARED((NPAD, D), jnp.float32),
        pltpu.VMEM_SHARED((NPAD,), jnp.float32),
        pltpu.SemaphoreType.DMA,
        pltpu.SemaphoreType.DMA,
    ],
)(_conv_xh_body)


_conv_hr = functools.partial(
    pl.kernel,
    out_type=jax.ShapeDtypeStruct((NC * NPAD, D), jnp.float32),
    mesh=plsc.VectorSubcoreMesh(core_axis_name="c", subcore_axis_name="s"),
    compiler_params=pltpu.CompilerParams(needs_layout_passes=False),
    scratch_types=[
        pltpu.VMEM((SCH, CH), jnp.int32),
        pltpu.VMEM((SCH, CH), jnp.int32),
        pltpu.VMEM((SCH, CH), jnp.float32),
        pltpu.VMEM((2, CH, D), jnp.float32),
        pltpu.VMEM_SHARED((NPAD, D), jnp.float32),
        pltpu.SemaphoreType.DMA,
    ],
)(_conv_hr_body)


def _gates_body(x_ref, h_ref, sx_ref, sh_ref, wx_ref, w1x_ref, wh_ref,
                w1h_ref, b_ref, z_ref, hr_ref, ph_ref):
    x = x_ref[...]
    h = h_ref[...]
    f32 = jnp.float32
    a = (jnp.dot(x, wx_ref[...], preferred_element_type=f32)
         + jnp.dot(sx_ref[...], w1x_ref[...], preferred_element_type=f32)
         + b_ref[...])
    bm = (jnp.dot(h, wh_ref[...], preferred_element_type=f32)
          + jnp.dot(sh_ref[...], w1h_ref[...], preferred_element_type=f32))
    z = jax.nn.sigmoid(a[:, :D] + bm[:, :D])
    r = jax.nn.sigmoid(a[:, D:2 * D] + bm[:, D:2 * D])
    z_ref[...] = z
    hr_ref[...] = h * r
    ph_ref[...] = a[:, 2 * D:]


def _out_body(ph_ref, hr_ref, s0_ref, s1_ref, z_ref, h_ref, whh_ref,
              w1hh_ref, wlin_ref, bhh_ref, blin_ref, out_ref, h0_ref):
    f32 = jnp.float32
    shr = s0_ref[...] + s1_ref[...]
    ht = jnp.tanh(ph_ref[...]
                  + jnp.dot(hr_ref[...], whh_ref[...], preferred_element_type=f32)
                  + jnp.dot(shr, w1hh_ref[...], preferred_element_type=f32)
                  + bhh_ref[...])
    z = z_ref[...]
    h0 = z * h_ref[...] + (1.0 - z) * ht
    h0_ref[...] = h0
    out_ref[...] = (jnp.dot(jnp.maximum(h0, 0.0), wlin_ref[...],
                            preferred_element_type=f32) + blin_ref[...])


_RB = 1000


def _row_spec(ncols):
    return pl.BlockSpec((_RB, ncols), lambda i: (i, 0))


def _full_spec(shape):
    nd = len(shape)
    return pl.BlockSpec(shape, lambda i: (0,) * nd)


def kernel(x, edge_index, edge_weight, h, params):
    p = params
    row = edge_index[0]
    col = edge_index[1]
    r4 = row.reshape(NS, NSUP, SCH, CH)
    c4 = col.reshape(NS, NSUP, SCH, CH)
    e4 = edge_weight.reshape(NS, NSUP, SCH, CH).astype(jnp.float32)
    zf = jnp.zeros((NPAD, D), jnp.float32)
    z1 = jnp.zeros((NPAD,), jnp.float32)

    sx, sh, lw = _conv_xh(x, h, r4, c4, e4, zf, z1)

    wx = jnp.concatenate([p["W0_xz"], p["W0_xr"], p["W0_xh"]], axis=1)
    w1x = jnp.concatenate([p["W1_xz"], p["W1_xr"], p["W1_xh"]], axis=1)
    wh = jnp.concatenate([p["W0_hz"], p["W0_hr"]], axis=1)
    w1h = jnp.concatenate([p["W1_hz"], p["W1_hr"]], axis=1)
    b384 = jnp.concatenate(
        [p["b_xz"] + p["b_hz"], p["b_xr"] + p["b_hr"], p["b_xh"]]).reshape(1, 3 * D)

    z, hr, ph = pl.pallas_call(
        _gates_body,
        grid=(N // _RB,),
        in_specs=[
            _row_spec(D), _row_spec(D), _row_spec(D), _row_spec(D),
            _full_spec((D, 3 * D)), _full_spec((D, 3 * D)),
            _full_spec((D, 2 * D)), _full_spec((D, 2 * D)),
            _full_spec((1, 3 * D)),
        ],
        out_specs=[_row_spec(D), _row_spec(D), _row_spec(D)],
        out_shape=[
            jax.ShapeDtypeStruct((N, D), jnp.float32),
            jax.ShapeDtypeStruct((N, D), jnp.float32),
            jax.ShapeDtypeStruct((N, D), jnp.float32),
        ],
    )(x, h, sx[:N], sh[:N], wx, w1x, wh, w1h, b384)

    shrp = _conv_hr(hr, row.reshape(NC * NS, NSUP2, SCH, CH),
                    col.reshape(NC * NS, NSUP2, SCH, CH),
                    lw.reshape(NC * NS, NSUP2, SCH, CH), zf)

    bhh = p["b_hh"].reshape(1, D)
    blin = p["b_lin"].reshape(1, D)
    out, h0 = pl.pallas_call(
        _out_body,
        grid=(N // _RB,),
        in_specs=[
            _row_spec(D), _row_spec(D), _row_spec(D), _row_spec(D),
            _row_spec(D), _row_spec(D),
            _full_spec((D, D)), _full_spec((D, D)), _full_spec((D, D)),
            _full_spec((1, D)), _full_spec((1, D)),
        ],
        out_specs=[_row_spec(D), _row_spec(D)],
        out_shape=[
            jax.ShapeDtypeStruct((N, D), jnp.float32),
            jax.ShapeDtypeStruct((N, D), jnp.float32),
        ],
    )(ph, hr, shrp[:N], shrp[NPAD:NPAD + N], z, h, p["W0_hh"], p["W1_hh"],
      p["W_lin"], bhh, blin)
    return (out, h0)


# speedup vs baseline: 15.5711x; 1.1092x over previous
import functools

import jax
import jax.numpy as jnp
from jax import lax
from jax.experimental import pallas as pl
from jax.experimental.pallas import tpu as pltpu
from jax.experimental.pallas import tpu_sc as plsc

N = 10000
E = 320000
D = 128
NC = 2
NS = 16
CH = 80
SCH = 25

NSUP = 10
NSUP2 = 5
NPAD = 10240
RPT = NPAD // NS


def _rsqrt_sc(d):
    i = lax.bitcast_convert_type(d, jnp.int32)
    y = lax.bitcast_convert_type(jnp.int32(0x5F3759DF) - (i >> 1), jnp.float32)
    for _ in range(3):
        t = (d * y) * y
        y = y * (1.5 - 0.5 * t)
    return jnp.where(d > 0.0, y, 0.0)


def _scale_rows(rows2, lwsup, par, cc):
    @plsc.parallel_loop(0, CH, unroll=4)
    def scale_body(e):
        lwb = plsc.load_gather(
            lwsup, [jnp.broadcast_to(cc, (16,)), jnp.broadcast_to(e, (16,))])
        for j in range(8):
            sl = pl.ds(j * 16, 16)
            rows2[par, e, sl] = rows2[par, e, sl] * lwb


def _spmm_super(table, rsup, csup, lwsup, rows2, acc, sem, sem2, compute_lw):
    pltpu.async_copy(table.at[rsup.at[0]], rows2.at[0], sem)

    def ch_body(cc, _):
        par = lax.rem(cc, 2)
        opp = lax.rem(cc + 1, 2)
        pltpu.make_async_copy(table.at[rsup.at[cc]], rows2.at[par], sem).wait()

        @pl.when(cc > 0)
        def _():
            pltpu.make_async_copy(rows2.at[opp], acc.at[csup.at[cc - 1]],
                                  sem2).wait()

        @pl.when(cc < SCH - 1)
        def _():
            pltpu.async_copy(table.at[rsup.at[cc + 1]], rows2.at[opp], sem)

        compute_lw(cc)
        _scale_rows(rows2, lwsup, par, cc)
        pltpu.async_copy(rows2.at[par], acc.at[csup.at[cc]], sem2, add=True)
        return 0

    lax.fori_loop(0, SCH, ch_body, 0)
    pltpu.make_async_copy(rows2.at[(SCH - 1) % 2], acc.at[csup.at[SCH - 1]],
                          sem2).wait()


def _conv_xh_body(x_hbm, h_hbm, r4, c4, e4, zf, z1, sx_o, sh_o, lw_o,
                  rsup, csup, esup, lwsup, dis_v, dtmp, rows2, acc, deg_sp,
                  sem, dsem):
    c = lax.axis_index("c")
    s = lax.axis_index("s")

    pltpu.sync_copy(z1.at[pl.ds(s * RPT, RPT)], deg_sp.at[pl.ds(s * RPT, RPT)])
    plsc.subcore_barrier()

    def deg_sup(sup, _):
        pltpu.sync_copy(r4.at[s, sup], rsup)
        pltpu.sync_copy(c4.at[s, sup], csup)
        pltpu.sync_copy(e4.at[s, sup], esup)

        def deg_ch(cc, _):
            for g in range(5):
                sl = pl.ds(g * 16, 16)
                esup[cc, sl] = jnp.where(
                    rsup[cc, sl] == csup[cc, sl], 0.0, esup[cc, sl])
            pltpu.async_copy(esup.at[cc], deg_sp.at[rsup.at[cc]], dsem,
                             add=True)
            return 0

        lax.fori_loop(0, SCH, deg_ch, 0)

        def deg_drain(cc, _):
            pltpu.make_async_copy(esup.at[cc], deg_sp.at[rsup.at[cc]],
                                  dsem).wait()
            return 0

        lax.fori_loop(0, SCH, deg_drain, 0)
        return 0

    lax.fori_loop(0, NSUP, deg_sup, 0)
    plsc.subcore_barrier()

    def dis_outer(k, _):
        pltpu.sync_copy(deg_sp.at[pl.ds(k * RPT, RPT)], dtmp)

        def dis_inner(v, _):
            d = dtmp[pl.ds(v * 16, 16)]
            dis_v[pl.ds(k * RPT + v * 16, 16)] = _rsqrt_sc(d)
            return 0

        lax.fori_loop(0, RPT // 16, dis_inner, 0)
        return 0

    lax.fori_loop(0, NS, dis_outer, 0)

    pltpu.sync_copy(zf.at[pl.ds(s * RPT, RPT)], acc.at[pl.ds(s * RPT, RPT)])
    plsc.subcore_barrier()

    def make_pass(table):
        def sup_body(sup, _):
            pltpu.sync_copy(r4.at[s, sup], rsup)
            pltpu.sync_copy(c4.at[s, sup], csup)
            pltpu.sync_copy(e4.at[s, sup], esup)

            def compute_lw(cc):
                for g in range(5):
                    sl = pl.ds(g * 16, 16)
                    r = rsup[cc, sl]
                    co = csup[cc, sl]
                    mew = jnp.where(r == co, 0.0, esup[cc, sl])
                    dr = plsc.load_gather(dis_v, [r])
                    dc = plsc.load_gather(dis_v, [co])
                    lwsup[cc, sl] = -(dr * mew * dc)

            _spmm_super(table, rsup, csup, lwsup, rows2, acc, sem, dsem,
                        compute_lw)

            @pl.when(c == 0)
            def _():
                pltpu.sync_copy(lwsup, lw_o.at[s, sup])

            return 0

        lax.fori_loop(0, NSUP, sup_body, 0)

    @pl.when(c == 0)
    def _():
        make_pass(x_hbm)

    @pl.when(c == 1)
    def _():
        make_pass(h_hbm)

    plsc.subcore_barrier()
    sl_out = pl.ds(s * RPT, RPT)

    @pl.when(c == 0)
    def _():
        pltpu.sync_copy(acc.at[sl_out], sx_o.at[sl_out])

    @pl.when(c == 1)
    def _():
        pltpu.sync_copy(acc.at[sl_out], sh_o.at[sl_out])


def _conv_hr_body(hr_hbm, r4, c4, lw4, zf, shr_o,
                  rsup, csup, lwsup, rows2, acc, sem, sem2):
    c = lax.axis_index("c")
    s = lax.axis_index("s")
    w = c * NS + s

    pltpu.sync_copy(zf.at[pl.ds(s * RPT, RPT)], acc.at[pl.ds(s * RPT, RPT)])
    plsc.subcore_barrier()

    def sup_body(sup, _):
        pltpu.sync_copy(r4.at[w, sup], rsup)
        pltpu.sync_copy(c4.at[w, sup], csup)
        pltpu.sync_copy(lw4.at[w, sup], lwsup)
        _spmm_super(hr_hbm, rsup, csup, lwsup, rows2, acc, sem, sem2,
                    lambda cc: None)
        return 0

    lax.fori_loop(0, NSUP2, sup_body, 0)
    plsc.subcore_barrier()
    pltpu.sync_copy(acc.at[pl.ds(s * RPT, RPT)],
                    shr_o.at[pl.ds(c * NPAD + s * RPT, RPT)])


_conv_xh = functools.partial(
    pl.kernel,
    out_type=(
        jax.ShapeDtypeStruct((NPAD, D), jnp.float32),
        jax.ShapeDtypeStruct((NPAD, D), jnp.float32),
        jax.ShapeDtypeStruct((NS, NSUP, SCH, CH), jnp.float32),
    ),
    mesh=plsc.VectorSubcoreMesh(core_axis_name="c", subcore_axis_name="s"),
    compiler_params=pltpu.CompilerParams(needs_layout_passes=False),
    scratch_types=[
        pltpu.VMEM((SCH, CH), jnp.int32),
        pltpu.VMEM((SCH, CH), jnp.int32),
        pltpu.VMEM((SCH, CH), jnp.float32),
        pltpu.VMEM((SCH, CH), jnp.float32),
        pltpu.VMEM((NPAD,), jnp.float32),
        pltpu.VMEM((RPT,), jnp.float32),
        pltpu.VMEM((2, CH, D), jnp.float32),
        pltpu.VMEM_SHARED((NPAD, D), jnp.float32),
        pltpu.VMEM_SHARED((NPAD,), jnp.float32),
        pltpu.SemaphoreType.DMA,
        pltpu.SemaphoreType.DMA,
    ],
)(_conv_xh_body)


_conv_hr = functools.partial(
    pl.kernel,
    out_type=jax.ShapeDtypeStruct((NC * NPAD, D), jnp.float32),
    mesh=plsc.VectorSubcoreMesh(core_axis_name="c", subcore_axis_name="s"),
    compiler_params=pltpu.CompilerParams(needs_layout_passes=False),
    scratch_types=[
        pltpu.VMEM((SCH, CH), jnp.int32),
        pltpu.VMEM((SCH, CH), jnp.int32),
        pltpu.VMEM((SCH, CH), jnp.float32),
        pltpu.VMEM((2, CH, D), jnp.float32),
        pltpu.VMEM_SHARED((NPAD, D), jnp.float32),
        pltpu.SemaphoreType.DMA,
        pltpu.SemaphoreType.DMA,
    ],
)(_conv_hr_body)


def _gates_body(x_ref, h_ref, sx_ref, sh_ref, wx_ref, w1x_ref, wh_ref,
                w1h_ref, b_ref, z_ref, hr_ref, ph_ref):
    x = x_ref[...]
    h = h_ref[...]
    f32 = jnp.float32
    a = (jnp.dot(x, wx_ref[...], preferred_element_type=f32)
         + jnp.dot(sx_ref[...], w1x_ref[...], preferred_element_type=f32)
         + b_ref[...])
    bm = (jnp.dot(h, wh_ref[...], preferred_element_type=f32)
          + jnp.dot(sh_ref[...], w1h_ref[...], preferred_element_type=f32))
    z = jax.nn.sigmoid(a[:, :D] + bm[:, :D])
    r = jax.nn.sigmoid(a[:, D:2 * D] + bm[:, D:2 * D])
    z_ref[...] = z
    hr_ref[...] = h * r
    ph_ref[...] = a[:, 2 * D:]


def _out_body(ph_ref, hr_ref, s0_ref, s1_ref, z_ref, h_ref, whh_ref,
              w1hh_ref, wlin_ref, bhh_ref, blin_ref, out_ref, h0_ref):
    f32 = jnp.float32
    shr = s0_ref[...] + s1_ref[...]
    ht = jnp.tanh(ph_ref[...]
                  + jnp.dot(hr_ref[...], whh_ref[...], preferred_element_type=f32)
                  + jnp.dot(shr, w1hh_ref[...], preferred_element_type=f32)
                  + bhh_ref[...])
    z = z_ref[...]
    h0 = z * h_ref[...] + (1.0 - z) * ht
    h0_ref[...] = h0
    out_ref[...] = (jnp.dot(jnp.maximum(h0, 0.0), wlin_ref[...],
                            preferred_element_type=f32) + blin_ref[...])


_RB = 1000


def _row_spec(ncols):
    return pl.BlockSpec((_RB, ncols), lambda i: (i, 0))


def _full_spec(shape):
    nd = len(shape)
    return pl.BlockSpec(shape, lambda i: (0,) * nd)


def kernel(x, edge_index, edge_weight, h, params):
    p = params
    row = edge_index[0]
    col = edge_index[1]
    r4 = row.reshape(NS, NSUP, SCH, CH)
    c4 = col.reshape(NS, NSUP, SCH, CH)
    e4 = edge_weight.reshape(NS, NSUP, SCH, CH).astype(jnp.float32)
    zf = jnp.zeros((NPAD, D), jnp.float32)
    z1 = jnp.zeros((NPAD,), jnp.float32)

    sx, sh, lw = _conv_xh(x, h, r4, c4, e4, zf, z1)

    wx = jnp.concatenate([p["W0_xz"], p["W0_xr"], p["W0_xh"]], axis=1)
    w1x = jnp.concatenate([p["W1_xz"], p["W1_xr"], p["W1_xh"]], axis=1)
    wh = jnp.concatenate([p["W0_hz"], p["W0_hr"]], axis=1)
    w1h = jnp.concatenate([p["W1_hz"], p["W1_hr"]], axis=1)
    b384 = jnp.concatenate(
        [p["b_xz"] + p["b_hz"], p["b_xr"] + p["b_hr"], p["b_xh"]]).reshape(1, 3 * D)

    z, hr, ph = pl.pallas_call(
        _gates_body,
        grid=(N // _RB,),
        in_specs=[
            _row_spec(D), _row_spec(D), _row_spec(D), _row_spec(D),
            _full_spec((D, 3 * D)), _full_spec((D, 3 * D)),
            _full_spec((D, 2 * D)), _full_spec((D, 2 * D)),
            _full_spec((1, 3 * D)),
        ],
        out_specs=[_row_spec(D), _row_spec(D), _row_spec(D)],
        out_shape=[
            jax.ShapeDtypeStruct((N, D), jnp.float32),
            jax.ShapeDtypeStruct((N, D), jnp.float32),
            jax.ShapeDtypeStruct((N, D), jnp.float32),
        ],
    )(x, h, sx[:N], sh[:N], wx, w1x, wh, w1h, b384)

    shrp = _conv_hr(hr, row.reshape(NC * NS, NSUP2, SCH, CH),
                    col.reshape(NC * NS, NSUP2, SCH, CH),
                    lw.reshape(NC * NS, NSUP2, SCH, CH), zf)

    bhh = p["b_hh"].reshape(1, D)
    blin = p["b_lin"].reshape(1, D)
    out, h0 = pl.pallas_call(
        _out_body,
        grid=(N // _RB,),
        in_specs=[
            _row_spec(D), _row_spec(D), _row_spec(D), _row_spec(D),
            _row_spec(D), _row_spec(D),
            _full_spec((D, D)), _full_spec((D, D)), _full_spec((D, D)),
            _full_spec((1, D)), _full_spec((1, D)),
        ],
        out_specs=[_row_spec(D), _row_spec(D)],
        out_shape=[
            jax.ShapeDtypeStruct((N, D), jnp.float32),
            jax.ShapeDtypeStruct((N, D), jnp.float32),
        ],
    )(ph, hr, shrp[:N], shrp[NPAD:NPAD + N], z, h, p["W0_hh"], p["W1_hh"],
      p["W_lin"], bhh, blin)
    return (out, h0)
